# Initial kernel scaffold; baseline (speedup 1.0000x reference)
import jax
import jax.numpy as jnp
from jax.experimental import pallas as pl


def kernel(token_ids, embdM):
    raise NotImplementedError("write your pallas kernel here")


# speedup vs baseline: 1.8731x; 1.8731x over previous
import functools

import jax
import jax.numpy as jnp
from jax import lax
from jax.experimental import pallas as pl
from jax.experimental.pallas import tpu as pltpu
from jax.experimental.pallas import tpu_sc as plsc

CHUNK = 128
K = 5


def _emb_kernel(n_total, d, per_w, n_chunks, n_groups):
    mesh = plsc.VectorSubcoreMesh(core_axis_name="c", subcore_axis_name="s")
    info = plsc.get_sparse_core_info()
    nc = info.num_cores

    grp_rows = K * CHUNK

    @functools.partial(
        pl.kernel,
        mesh=mesh,
        compiler_params=pltpu.CompilerParams(use_tc_tiling_on_sc=False),
        out_type=jax.ShapeDtypeStruct((n_total, d), jnp.float32),
        scratch_types=[
            pltpu.VMEM((n_chunks, CHUNK), jnp.int32),
            pltpu.VMEM((2, grp_rows, d), jnp.float32),
            pltpu.SemaphoreType.DMA,
            pltpu.SemaphoreType.DMA,
        ],
    )
    def emb(idx_hbm, table_hbm, out_hbm, idx_v, rows_v, gsem, wsem):
        wid = lax.axis_index("s") * nc + lax.axis_index("c")
        base = wid * per_w

        pltpu.sync_copy(idx_hbm.at[wid], idx_v)

        def fire_gathers(g, bank):
            for j in range(K):
                c = g * K + j
                pltpu.make_async_copy(
                    table_hbm.at[idx_v.at[c]],
                    rows_v.at[bank, pl.ds(j * CHUNK, CHUNK)],
                    gsem,
                ).start()

        def drain_gathers(bank):
            for j in range(K):
                pltpu.make_async_copy(
                    table_hbm.at[idx_v.at[0]],
                    rows_v.at[bank, pl.ds(j * CHUNK, CHUNK)],
                    gsem,
                ).wait()

        fire_gathers(0, 0)

        def body(g, _):
            bank = lax.rem(g, 2)
            drain_gathers(bank)
            pltpu.make_async_copy(
                rows_v.at[bank],
                out_hbm.at[pl.ds(base + g * grp_rows, grp_rows)],
                wsem,
            ).start()

            @pl.when(g > 0)
            def _():
                pltpu.make_async_copy(
                    rows_v.at[1 - bank],
                    out_hbm.at[pl.ds(base, grp_rows)],
                    wsem,
                ).wait()

            @pl.when(g + 1 < n_groups)
            def _():
                fire_gathers(g + 1, 1 - bank)

            return 0

        lax.fori_loop(0, n_groups, body, 0)

        pltpu.make_async_copy(
            rows_v.at[0],
            out_hbm.at[pl.ds(base, grp_rows)],
            wsem,
        ).wait()

    return emb


def kernel(token_ids, embdM):
    b, s = token_ids.shape
    n_total = b * s
    d = embdM.shape[1]
    info = plsc.get_sparse_core_info()
    nw = info.num_cores * info.num_subcores
    per_w = n_total // nw
    n_chunks = per_w // CHUNK
    n_groups = n_chunks // K

    idx = token_ids.reshape(nw, n_chunks, CHUNK).astype(jnp.int32)
    out = _emb_kernel(n_total, d, per_w, n_chunks, n_groups)(idx, embdM)
    return out.reshape(b, s, d)


# speedup vs baseline: 1.8739x; 1.0004x over previous
import functools

import jax
import jax.numpy as jnp
from jax import lax
from jax.experimental import pallas as pl
from jax.experimental.pallas import tpu as pltpu
from jax.experimental.pallas import tpu_sc as plsc

CHUNK = 128
K = 5


def _emb_kernel(n_total, d, per_w, n_chunks, n_groups):
    mesh = plsc.VectorSubcoreMesh(core_axis_name="c", subcore_axis_name="s")
    info = plsc.get_sparse_core_info()
    nc = info.num_cores

    grp_rows = K * CHUNK

    @functools.partial(
        pl.kernel,
        mesh=mesh,
        compiler_params=pltpu.CompilerParams(use_tc_tiling_on_sc=False),
        out_type=jax.ShapeDtypeStruct((n_total, d), jnp.float32),
        scratch_types=[
            pltpu.VMEM((n_chunks, CHUNK), jnp.int32),
            pltpu.VMEM((2, grp_rows, d), jnp.float32),
            pltpu.SemaphoreType.DMA,
            pltpu.SemaphoreType.DMA,
        ],
    )
    def emb(idx_hbm, table_hbm, out_hbm, idx_v, rows_v, gsem, wsem):
        wid = lax.axis_index("s") * nc + lax.axis_index("c")
        base = wid * per_w

        pltpu.sync_copy(idx_hbm.at[wid], idx_v)

        def fire_gathers(g, bank):
            for j in range(K):
                c = g * K + j
                pltpu.make_async_copy(
                    table_hbm.at[idx_v.at[c]],
                    rows_v.at[bank, pl.ds(j * CHUNK, CHUNK)],
                    gsem,
                ).start()

        def drain_gathers(bank):
            for j in range(K):
                pltpu.make_async_copy(
                    table_hbm.at[idx_v.at[0]],
                    rows_v.at[bank, pl.ds(j * CHUNK, CHUNK)],
                    gsem,
                ).wait()

        fire_gathers(0, 0)

        def body(g, _):
            bank = lax.rem(g, 2)

            @pl.when(g > 0)
            def _():
                pltpu.make_async_copy(
                    rows_v.at[1 - bank],
                    out_hbm.at[pl.ds(base, grp_rows)],
                    wsem,
                ).wait()

            @pl.when(g + 1 < n_groups)
            def _():
                fire_gathers(g + 1, 1 - bank)

            drain_gathers(bank)
            pltpu.make_async_copy(
                rows_v.at[bank],
                out_hbm.at[pl.ds(base + g * grp_rows, grp_rows)],
                wsem,
            ).start()
            return 0

        lax.fori_loop(0, n_groups, body, 0)

        pltpu.make_async_copy(
            rows_v.at[0],
            out_hbm.at[pl.ds(base, grp_rows)],
            wsem,
        ).wait()

    return emb


def kernel(token_ids, embdM):
    b, s = token_ids.shape
    n_total = b * s
    d = embdM.shape[1]
    info = plsc.get_sparse_core_info()
    nw = info.num_cores * info.num_subcores
    per_w = n_total // nw
    n_chunks = per_w // CHUNK
    n_groups = n_chunks // K

    idx = token_ids.reshape(nw, n_chunks, CHUNK).astype(jnp.int32)
    out = _emb_kernel(n_total, d, per_w, n_chunks, n_groups)(idx, embdM)
    return out.reshape(b, s, d)


# speedup vs baseline: 2.5172x; 1.3433x over previous
import functools

import jax
import jax.numpy as jnp
from jax import lax
from jax.experimental import pallas as pl
from jax.experimental.pallas import tpu as pltpu
from jax.experimental.pallas import tpu_sc as plsc

GRP = 8


def _emb_kernel(b, s, d, per_w):
    mesh = plsc.VectorSubcoreMesh(core_axis_name="c", subcore_axis_name="s")
    info = plsc.get_sparse_core_info()
    nc = info.num_cores
    n_groups = per_w // GRP
    s_pad = (s + 7) // 8 * 8
    d_pad = (d + 127) // 128 * 128

    @functools.partial(
        pl.kernel,
        mesh=mesh,
        compiler_params=pltpu.CompilerParams(use_tc_tiling_on_sc=False),
        out_type=jax.ShapeDtypeStruct((b, s_pad, d_pad), jnp.float32),
        scratch_types=[
            pltpu.VMEM((2, GRP, s), jnp.int32),
            pltpu.VMEM((2, GRP, s, d), jnp.float32),
            pltpu.SemaphoreType.DMA,
            pltpu.SemaphoreType.DMA,
            pltpu.SemaphoreType.DMA,
        ],
    )
    def emb(ids_hbm, table_hbm, out_hbm, idx_v, rows_v, isem, gsem, wsem):
        wid = lax.axis_index("s") * nc + lax.axis_index("c")
        tok0 = wid * per_w

        def stage_idx(g, bank):
            pltpu.make_async_copy(
                ids_hbm.at[pl.ds(tok0 + g * GRP, GRP)], idx_v.at[bank], isem
            ).start()

        def wait_idx(bank):
            pltpu.make_async_copy(
                ids_hbm.at[pl.ds(tok0, GRP)], idx_v.at[bank], isem
            ).wait()

        def fire_gathers(bank):
            for t in range(GRP):
                pltpu.make_async_copy(
                    table_hbm.at[idx_v.at[bank, t]], rows_v.at[bank, t], gsem
                ).start()

        def drain_gathers(bank):
            for t in range(GRP):
                pltpu.make_async_copy(
                    table_hbm.at[idx_v.at[bank, t]], rows_v.at[bank, t], gsem
                ).wait()

        stage_idx(0, 0)
        stage_idx(1, 1)
        wait_idx(0)
        fire_gathers(0)

        def body(g, _):
            bank = lax.rem(g, 2)

            @pl.when(g > 0)
            def _():
                pltpu.make_async_copy(
                    rows_v.at[1 - bank],
                    out_hbm.at[pl.ds(tok0, GRP), pl.ds(0, s), pl.ds(0, d)],
                    wsem,
                ).wait()

            @pl.when(g + 1 < n_groups)
            def _():
                wait_idx(1 - bank)
                fire_gathers(1 - bank)

            drain_gathers(bank)

            @pl.when(g + 2 < n_groups)
            def _():
                stage_idx(g + 2, bank)

            pltpu.make_async_copy(
                rows_v.at[bank],
                out_hbm.at[pl.ds(tok0 + g * GRP, GRP), pl.ds(0, s), pl.ds(0, d)],
                wsem,
            ).start()
            return 0

        lax.fori_loop(0, n_groups, body, 0)

        pltpu.make_async_copy(
            rows_v.at[(n_groups - 1) % 2],
            out_hbm.at[pl.ds(tok0, GRP), pl.ds(0, s), pl.ds(0, d)],
            wsem,
        ).wait()

    return emb


def kernel(token_ids, embdM):
    b, s = token_ids.shape
    d = embdM.shape[1]
    info = plsc.get_sparse_core_info()
    nw = info.num_cores * info.num_subcores
    per_w = b // nw

    out_p = _emb_kernel(b, s, d, per_w)(token_ids.astype(jnp.int32), embdM)
    return out_p[:, :s, :d]
